# baseline (speedup 1.0000x reference)
import functools

import jax
import jax.numpy as jnp
from jax import lax
from jax.experimental import pallas as pl
from jax.experimental.pallas import tpu as pltpu
from jax.experimental.pallas import tpu_sc as plsc

NSEG = 256
NROWS = 100000
D = 768
LANES = 128
CT = D // LANES
NGRP = NROWS // 8
NC, NS = 2, 16
NW = NC * NS
CHUNK = 32
GPC = CHUNK // 8
NCH = NROWS // CHUNK
NKMAX = -(-NCH // NW)
NFULL = NCH - NW * (NKMAX - 1)
NBUF = 3
CW = 16
ACC = NSEG * CT
ARS = ACC // NS
RS = NSEG // NS


def _sc_segment_sum(xp, batch2, pidx3, ones, zsum, zcnt):
    mesh = plsc.VectorSubcoreMesh(
        core_axis_name="c", subcore_axis_name="s",
        num_cores=NC, num_subcores=NS)

    @functools.partial(
        pl.kernel,
        out_type=[
            jax.ShapeDtypeStruct((NC, ACC, LANES), jnp.float32),
            jax.ShapeDtypeStruct((NC, NSEG, CW), jnp.float32),
        ],
        mesh=mesh,
        scratch_types=[
            pltpu.VMEM((NKMAX, CHUNK), jnp.int32),
            pltpu.VMEM((NKMAX, 2, CHUNK * CT // 2), jnp.int32),
            pltpu.VMEM((NBUF, CHUNK * CT, LANES), jnp.float32),
            pltpu.VMEM((CHUNK, CW), jnp.float32),
            pltpu.VMEM_SHARED((ACC, LANES), jnp.float32),
            pltpu.VMEM_SHARED((NSEG, CW), jnp.float32),
            pltpu.SemaphoreType.DMA((NBUF,)),
            pltpu.SemaphoreType.DMA((NBUF,)),
            pltpu.SemaphoreType.DMA((NBUF,)),
        ],
        compiler_params=pltpu.CompilerParams(use_tc_tiling_on_sc=False),
    )
    def body(xp_hbm, b2_hbm, pidx_hbm, ones_hbm, zs_hbm, zc_hbm,
             sums_out, cnt_out,
             idx_v, pidx_v, bufs, ones_v, acc_s, acc_c,
             gsem, ssem, csem):
        c = lax.axis_index("c")
        s = lax.axis_index("s")
        wid = c * NS + s
        nk = jnp.where(wid < NFULL, NKMAX, NKMAX - 1)
        c0 = wid * (NKMAX - 1) + jnp.minimum(wid, NFULL)

        pltpu.sync_copy(zs_hbm, acc_s.at[pl.ds(s * ARS, ARS)])
        pltpu.sync_copy(zc_hbm, acc_c.at[pl.ds(s * RS, RS)])
        pltpu.sync_copy(b2_hbm.at[pl.ds(c0, NKMAX - 1)],
                        idx_v.at[pl.ds(0, NKMAX - 1)])
        pltpu.sync_copy(pidx_hbm.at[pl.ds(c0, NKMAX - 1)],
                        pidx_v.at[pl.ds(0, NKMAX - 1)])

        @pl.when(nk == NKMAX)
        def _():
            pltpu.sync_copy(b2_hbm.at[pl.ds(c0 + NKMAX - 1, 1)],
                            idx_v.at[pl.ds(NKMAX - 1, 1)])
            pltpu.sync_copy(pidx_hbm.at[pl.ds(c0 + NKMAX - 1, 1)],
                            pidx_v.at[pl.ds(NKMAX - 1, 1)])

        pltpu.sync_copy(ones_hbm, ones_v)
        plsc.subcore_barrier()

        HALF = CHUNK * CT // 2

        def start_gather(j, b):
            p0 = (c0 + j) * CHUNK * CT
            pltpu.async_copy(xp_hbm.at[pl.ds(p0, CHUNK * CT)],
                             bufs.at[b], gsem.at[b])

        for j0 in range(NBUF - 2):
            start_gather(j0, j0)

        def wait_gather(b):
            pltpu.make_async_copy(xp_hbm.at[pl.ds(0, CHUNK * CT)],
                                  bufs.at[b], gsem.at[b]).wait()

        def wait_scatter(b):
            for h in range(2):
                pltpu.make_async_copy(bufs.at[b, pl.ds(h * HALF, HALF)],
                                      acc_s.at[pidx_v.at[0, 0]],
                                      ssem.at[b]).wait()
            pltpu.make_async_copy(ones_v, acc_c.at[idx_v.at[0]],
                                  csem.at[b]).wait()

        def process_chunk(k, b):
            wait_gather(b)
            for h in range(2):
                pltpu.async_copy(bufs.at[b, pl.ds(h * HALF, HALF)],
                                 acc_s.at[pidx_v.at[k, h]],
                                 ssem.at[b], add=True)
            pltpu.async_copy(ones_v, acc_c.at[idx_v.at[k]], csem.at[b],
                             add=True)
            j = k + (NBUF - 2)
            bj = (b + NBUF - 2) % NBUF

            @pl.when(j < nk)
            def _():
                @pl.when(j >= NBUF)
                def _():
                    wait_scatter(bj)
                start_gather(j, bj)

        def outer_body(it, carry):
            for b in range(NBUF):
                k = it * NBUF + b

                @pl.when(k < nk)
                def _(k=k, b=b):
                    process_chunk(k, b)

            return carry

        lax.fori_loop(0, (nk + NBUF - 1) // NBUF, outer_body, 0)
        for b in range(NBUF):
            wait_scatter(b)
        plsc.subcore_barrier()

        pltpu.sync_copy(acc_s.at[pl.ds(s * ARS, ARS)],
                        sums_out.at[c, pl.ds(s * ARS, ARS)])
        pltpu.sync_copy(acc_c.at[pl.ds(s * RS, RS)],
                        cnt_out.at[c, pl.ds(s * RS, RS)])

    return body(xp, batch2, pidx3, ones, zsum, zcnt)


def _tc_head(sums4, cnt2, W1, b1r, W2, b2r):
    def body(s_ref, c_ref, w1_ref, b1_ref, w2_ref, b2_ref, out_ref):
        sums = jnp.concatenate(
            [s_ref[0, t] + s_ref[1, t] for t in range(CT)], axis=1)
        cnt = c_ref[0, :, 0:1] + c_ref[1, :, 0:1]
        pooled = sums / jnp.clip(cnt, 1.0, None)
        h = lax.dot_general(
            pooled, w1_ref[...],
            dimension_numbers=(((1,), (1,)), ((), ())),
            preferred_element_type=jnp.float32,
            precision=lax.Precision.HIGHEST)
        h = jnp.maximum(h + b1_ref[...], 0.0)
        o = lax.dot_general(
            h, w2_ref[...],
            dimension_numbers=(((1,), (1,)), ((), ())),
            preferred_element_type=jnp.float32,
            precision=lax.Precision.HIGHEST)
        out_ref[...] = o[:, 0:1] + b2_ref[0, 0]

    return pl.pallas_call(
        body,
        in_specs=[
            pl.BlockSpec(memory_space=pltpu.VMEM),
            pl.BlockSpec(memory_space=pltpu.VMEM),
            pl.BlockSpec(memory_space=pltpu.VMEM),
            pl.BlockSpec(memory_space=pltpu.VMEM),
            pl.BlockSpec(memory_space=pltpu.VMEM),
            pl.BlockSpec(memory_space=pltpu.SMEM),
        ],
        out_shape=jax.ShapeDtypeStruct((NSEG, 1), jnp.float32),
    )(sums4, cnt2, W1, b1r, W2, b2r)


@jax.jit
def kernel(x, batch, W1, b1, W2, b2):
    xp = (x.reshape(NGRP, 8, CT, LANES)
          .transpose(0, 2, 1, 3)
          .reshape(NGRP * CT * 8, LANES))
    batch2 = batch.astype(jnp.int32).reshape(NCH, CHUNK)
    pidx = jnp.concatenate(
        [batch2[:, g * 8:(g + 1) * 8] + (t * NSEG)
         for g in range(GPC) for t in range(CT)], axis=1)
    pidx3 = pidx.reshape(NCH, 2, CHUNK * CT // 2)
    ones = jnp.ones((CHUNK, CW), jnp.float32)
    zs = jnp.zeros((ARS, LANES), jnp.float32)
    zc = jnp.zeros((RS, CW), jnp.float32)
    sums2, cnt2 = _sc_segment_sum(xp, batch2, pidx3, ones, zs, zc)
    sums4 = sums2.reshape(NC, CT, NSEG, LANES)
    W2p = jnp.pad(W2, ((0, 7), (0, 0)))
    out = _tc_head(sums4, cnt2, W1, b1.reshape(1, 128), W2p, b2.reshape(1, 1))
    return out[:, 0]


# speedup vs baseline: 1.0485x; 1.0485x over previous
import functools

import jax
import jax.numpy as jnp
from jax import lax
from jax.experimental import pallas as pl
from jax.experimental.pallas import tpu as pltpu
from jax.experimental.pallas import tpu_sc as plsc

NSEG = 256
NROWS = 100000
D = 768
LANES = 128
CT = D // LANES
NGRP = NROWS // 8
NC, NS = 2, 16
NW = NC * NS
CHUNK = 32
GPC = CHUNK // 8
NCH = NROWS // CHUNK
NKMAX = -(-NCH // NW)
NFULL = NCH - NW * (NKMAX - 1)
NBUF = 3
CW = 16
ACC = NSEG * CT
ARS = ACC // NS
RS = NSEG // NS


def _sc_segment_sum(xp, batch2, pidx3, ones, zsum, zcnt):
    mesh = plsc.VectorSubcoreMesh(
        core_axis_name="c", subcore_axis_name="s",
        num_cores=NC, num_subcores=NS)

    @functools.partial(
        pl.kernel,
        out_type=[
            jax.ShapeDtypeStruct((NC, ACC, LANES), jnp.float32),
            jax.ShapeDtypeStruct((NC, NSEG, CW), jnp.float32),
        ],
        mesh=mesh,
        scratch_types=[
            pltpu.VMEM((NKMAX, CHUNK), jnp.int32),
            pltpu.VMEM((NKMAX, 2, CHUNK * CT // 2), jnp.int32),
            pltpu.VMEM((NBUF, CHUNK * CT, LANES), jnp.float32),
            pltpu.VMEM((CHUNK, CW), jnp.float32),
            pltpu.VMEM_SHARED((ACC, LANES), jnp.float32),
            pltpu.VMEM_SHARED((NSEG, CW), jnp.float32),
            pltpu.SemaphoreType.DMA((NBUF,)),
            pltpu.SemaphoreType.DMA((NBUF,)),
            pltpu.SemaphoreType.DMA((NBUF,)),
        ],
        compiler_params=pltpu.CompilerParams(use_tc_tiling_on_sc=False),
    )
    def body(xp_hbm, b2_hbm, pidx_hbm, ones_hbm, zs_hbm, zc_hbm,
             sums_out, cnt_out,
             idx_v, pidx_v, bufs, ones_v, acc_s, acc_c,
             gsem, ssem, csem):
        c = lax.axis_index("c")
        s = lax.axis_index("s")
        wid = c * NS + s
        nk = jnp.where(wid < NFULL, NKMAX, NKMAX - 1)
        c0 = wid * (NKMAX - 1) + jnp.minimum(wid, NFULL)

        pltpu.sync_copy(zs_hbm, acc_s.at[pl.ds(s * ARS, ARS)])
        pltpu.sync_copy(zc_hbm, acc_c.at[pl.ds(s * RS, RS)])
        pltpu.sync_copy(b2_hbm.at[pl.ds(c0, NKMAX - 1)],
                        idx_v.at[pl.ds(0, NKMAX - 1)])
        pltpu.sync_copy(pidx_hbm.at[pl.ds(c0, NKMAX - 1)],
                        pidx_v.at[pl.ds(0, NKMAX - 1)])

        @pl.when(nk == NKMAX)
        def _():
            pltpu.sync_copy(b2_hbm.at[pl.ds(c0 + NKMAX - 1, 1)],
                            idx_v.at[pl.ds(NKMAX - 1, 1)])
            pltpu.sync_copy(pidx_hbm.at[pl.ds(c0 + NKMAX - 1, 1)],
                            pidx_v.at[pl.ds(NKMAX - 1, 1)])

        pltpu.sync_copy(ones_hbm, ones_v)
        plsc.subcore_barrier()

        HALF = CHUNK * CT // 2

        def start_gather(j, b):
            p0 = (c0 + j) * CHUNK * CT
            pltpu.async_copy(xp_hbm.at[pl.ds(p0, CHUNK * CT)],
                             bufs.at[b], gsem.at[b])

        for j0 in range(NBUF - 2):
            start_gather(j0, j0)

        def wait_gather(b):
            pltpu.make_async_copy(xp_hbm.at[pl.ds(0, CHUNK * CT)],
                                  bufs.at[b], gsem.at[b]).wait()

        def wait_scatter(b):
            for h in range(2):
                pltpu.make_async_copy(bufs.at[b, pl.ds(h * HALF, HALF)],
                                      acc_s.at[pidx_v.at[0, 0]],
                                      ssem.at[b]).wait()
            pltpu.make_async_copy(ones_v, acc_c.at[idx_v.at[0]],
                                  csem.at[b]).wait()

        def process_chunk(k, b):
            wait_gather(b)
            for h in range(2):
                pltpu.async_copy(bufs.at[b, pl.ds(h * HALF, HALF)],
                                 acc_s.at[pidx_v.at[k, h]],
                                 ssem.at[b], add=True)
            pltpu.async_copy(ones_v, acc_c.at[idx_v.at[k]], csem.at[b],
                             add=True)
            j = k + (NBUF - 2)
            bj = (b + NBUF - 2) % NBUF

            @pl.when(j < nk)
            def _():
                @pl.when(j >= NBUF)
                def _():
                    wait_scatter(bj)
                start_gather(j, bj)

        def outer_body(it, carry):
            for b in range(NBUF):
                k = it * NBUF + b

                @pl.when(k < nk)
                def _(k=k, b=b):
                    process_chunk(k, b)

            return carry

        lax.fori_loop(0, (nk + NBUF - 1) // NBUF, outer_body, 0)
        for b in range(NBUF):
            wait_scatter(b)
        plsc.subcore_barrier()

        pltpu.sync_copy(acc_s.at[pl.ds(s * ARS, ARS)],
                        sums_out.at[c, pl.ds(s * ARS, ARS)])
        pltpu.sync_copy(acc_c.at[pl.ds(s * RS, RS)],
                        cnt_out.at[c, pl.ds(s * RS, RS)])

    return body(xp, batch2, pidx3, ones, zsum, zcnt)


def _tc_head(sums4, cnt2, W1, b1r, W2, b2r):
    def body(s_ref, c_ref, w1_ref, b1_ref, w2_ref, b2_ref, out_ref):
        sums = jnp.concatenate(
            [s_ref[0, t] + s_ref[1, t] for t in range(CT)], axis=1)
        cnt = c_ref[0, :, 0:1] + c_ref[1, :, 0:1]
        pooled = sums / jnp.clip(cnt, 1.0, None)
        h = lax.dot_general(
            pooled, w1_ref[...],
            dimension_numbers=(((1,), (1,)), ((), ())),
            preferred_element_type=jnp.float32,
            precision=lax.Precision.HIGHEST)
        h = jnp.maximum(h + b1_ref[...], 0.0)
        o = lax.dot_general(
            h, w2_ref[...],
            dimension_numbers=(((1,), (1,)), ((), ())),
            preferred_element_type=jnp.float32,
            precision=lax.Precision.HIGHEST)
        out_ref[...] = o[:, 0:1] + b2_ref[0, 0]

    return pl.pallas_call(
        body,
        in_specs=[
            pl.BlockSpec(memory_space=pltpu.VMEM),
            pl.BlockSpec(memory_space=pltpu.VMEM),
            pl.BlockSpec(memory_space=pltpu.VMEM),
            pl.BlockSpec(memory_space=pltpu.VMEM),
            pl.BlockSpec(memory_space=pltpu.VMEM),
            pl.BlockSpec(memory_space=pltpu.SMEM),
        ],
        out_shape=jax.ShapeDtypeStruct((NSEG, 1), jnp.float32),
    )(sums4, cnt2, W1, b1r, W2, b2r)


@jax.jit
def kernel(x, batch, W1, b1, W2, b2):
    xp = (x.reshape(NGRP, 8, CT, LANES)
          .transpose(0, 2, 1, 3)
          .reshape(NGRP * CT * 8, LANES))
    batch2 = batch.astype(jnp.int32).reshape(NCH, CHUNK)
    pidx = (batch2.reshape(NCH, GPC, 1, 8)
            + (NSEG * jnp.arange(CT, dtype=jnp.int32)).reshape(1, 1, CT, 1))
    pidx3 = pidx.reshape(NCH, 2, CHUNK * CT // 2)
    ones = jnp.ones((CHUNK, CW), jnp.float32)
    zs = jnp.zeros((ARS, LANES), jnp.float32)
    zc = jnp.zeros((RS, CW), jnp.float32)
    sums2, cnt2 = _sc_segment_sum(xp, batch2, pidx3, ones, zs, zc)
    sums4 = sums2.reshape(NC, CT, NSEG, LANES)
    W2p = jnp.pad(W2, ((0, 7), (0, 0)))
    out = _tc_head(sums4, cnt2, W1, b1.reshape(1, 128), W2p, b2.reshape(1, 1))
    return out[:, 0]


# speedup vs baseline: 1.0493x; 1.0007x over previous
import functools

import jax
import jax.numpy as jnp
from jax import lax
from jax.experimental import pallas as pl
from jax.experimental.pallas import tpu as pltpu
from jax.experimental.pallas import tpu_sc as plsc

NSEG = 256
NROWS = 100000
D = 768
LANES = 128
CT = D // LANES
NGRP = NROWS // 8
NC, NS = 2, 16
NW = NC * NS
CHUNK = 32
GPC = CHUNK // 8
NCH = NROWS // CHUNK
NKMAX = -(-NCH // NW)
NFULL = NCH - NW * (NKMAX - 1)
NBUF = 3
CW = 16
ACC = NSEG * CT
ARS = ACC // NS
RS = NSEG // NS


def _sc_segment_sum(xp, batch2, pidx3, ones, zsum, zcnt):
    mesh = plsc.VectorSubcoreMesh(
        core_axis_name="c", subcore_axis_name="s",
        num_cores=NC, num_subcores=NS)

    @functools.partial(
        pl.kernel,
        out_type=[
            jax.ShapeDtypeStruct((NC, ACC, LANES), jnp.float32),
            jax.ShapeDtypeStruct((NC, NSEG, CW), jnp.float32),
        ],
        mesh=mesh,
        scratch_types=[
            pltpu.VMEM((NKMAX, CHUNK), jnp.int32),
            pltpu.VMEM((NKMAX, 2, CHUNK * CT // 2), jnp.int32),
            pltpu.VMEM((NBUF, CHUNK * CT, LANES), jnp.float32),
            pltpu.VMEM((CHUNK, CW), jnp.float32),
            pltpu.VMEM_SHARED((ACC, LANES), jnp.float32),
            pltpu.VMEM_SHARED((NSEG, CW), jnp.float32),
            pltpu.SemaphoreType.DMA((NBUF,)),
            pltpu.SemaphoreType.DMA((NBUF,)),
            pltpu.SemaphoreType.DMA((NBUF,)),
        ],
        compiler_params=pltpu.CompilerParams(use_tc_tiling_on_sc=False),
    )
    def body(xp_hbm, b2_hbm, pidx_hbm, ones_hbm, zs_hbm, zc_hbm,
             sums_out, cnt_out,
             idx_v, pidx_v, bufs, ones_v, acc_s, acc_c,
             gsem, ssem, csem):
        c = lax.axis_index("c")
        s = lax.axis_index("s")
        wid = c * NS + s
        nk = jnp.where(wid < NFULL, NKMAX, NKMAX - 1)
        c0 = wid * (NKMAX - 1) + jnp.minimum(wid, NFULL)

        pltpu.sync_copy(zs_hbm, acc_s.at[pl.ds(s * ARS, ARS)])
        pltpu.sync_copy(zc_hbm, acc_c.at[pl.ds(s * RS, RS)])
        pltpu.sync_copy(b2_hbm.at[pl.ds(c0, NKMAX - 1)],
                        idx_v.at[pl.ds(0, NKMAX - 1)])
        pltpu.sync_copy(pidx_hbm.at[pl.ds(c0, NKMAX - 1)],
                        pidx_v.at[pl.ds(0, NKMAX - 1)])

        @pl.when(nk == NKMAX)
        def _():
            pltpu.sync_copy(b2_hbm.at[pl.ds(c0 + NKMAX - 1, 1)],
                            idx_v.at[pl.ds(NKMAX - 1, 1)])
            pltpu.sync_copy(pidx_hbm.at[pl.ds(c0 + NKMAX - 1, 1)],
                            pidx_v.at[pl.ds(NKMAX - 1, 1)])

        pltpu.sync_copy(ones_hbm, ones_v)
        plsc.subcore_barrier()

        HALF = CHUNK * CT // 2

        def start_gather(j, b):
            p0 = (c0 + j) * CHUNK * CT
            pltpu.async_copy(xp_hbm.at[pl.ds(p0, CHUNK * CT)],
                             bufs.at[b], gsem.at[b])

        for j0 in range(NBUF - 2):
            start_gather(j0, j0)

        def wait_gather(b):
            pltpu.make_async_copy(xp_hbm.at[pl.ds(0, CHUNK * CT)],
                                  bufs.at[b], gsem.at[b]).wait()

        def wait_scatter(b):
            for h in range(2):
                pltpu.make_async_copy(bufs.at[b, pl.ds(h * HALF, HALF)],
                                      acc_s.at[pidx_v.at[0, 0]],
                                      ssem.at[b]).wait()
            pltpu.make_async_copy(ones_v, acc_c.at[idx_v.at[0]],
                                  csem.at[b]).wait()

        def chunk_body(k, carry):
            b = lax.rem(k, NBUF)
            wait_gather(b)
            for h in range(2):
                pltpu.async_copy(bufs.at[b, pl.ds(h * HALF, HALF)],
                                 acc_s.at[pidx_v.at[k, h]],
                                 ssem.at[b], add=True)
            pltpu.async_copy(ones_v, acc_c.at[idx_v.at[k]], csem.at[b],
                             add=True)
            j = k + (NBUF - 2)
            bj = lax.rem(j, NBUF)

            @pl.when(j < nk)
            def _():
                @pl.when(j >= NBUF)
                def _():
                    wait_scatter(bj)
                start_gather(j, bj)

            return carry

        lax.fori_loop(0, nk, chunk_body, 0)
        for b in range(NBUF):
            wait_scatter(b)
        plsc.subcore_barrier()

        pltpu.sync_copy(acc_s.at[pl.ds(s * ARS, ARS)],
                        sums_out.at[c, pl.ds(s * ARS, ARS)])
        pltpu.sync_copy(acc_c.at[pl.ds(s * RS, RS)],
                        cnt_out.at[c, pl.ds(s * RS, RS)])

    return body(xp, batch2, pidx3, ones, zsum, zcnt)


def _tc_head(sums4, cnt2, W1, b1r, W2, b2r):
    def body(s_ref, c_ref, w1_ref, b1_ref, w2_ref, b2_ref, out_ref):
        sums = jnp.concatenate(
            [s_ref[0, t] + s_ref[1, t] for t in range(CT)], axis=1)
        cnt = c_ref[0, :, 0:1] + c_ref[1, :, 0:1]
        pooled = sums / jnp.clip(cnt, 1.0, None)
        h = lax.dot_general(
            pooled, w1_ref[...],
            dimension_numbers=(((1,), (1,)), ((), ())),
            preferred_element_type=jnp.float32,
            precision=lax.Precision.HIGHEST)
        h = jnp.maximum(h + b1_ref[...], 0.0)
        o = lax.dot_general(
            h, w2_ref[...],
            dimension_numbers=(((1,), (1,)), ((), ())),
            preferred_element_type=jnp.float32,
            precision=lax.Precision.HIGHEST)
        out_ref[...] = o[:, 0:1] + b2_ref[0, 0]

    return pl.pallas_call(
        body,
        in_specs=[
            pl.BlockSpec(memory_space=pltpu.VMEM),
            pl.BlockSpec(memory_space=pltpu.VMEM),
            pl.BlockSpec(memory_space=pltpu.VMEM),
            pl.BlockSpec(memory_space=pltpu.VMEM),
            pl.BlockSpec(memory_space=pltpu.VMEM),
            pl.BlockSpec(memory_space=pltpu.SMEM),
        ],
        out_shape=jax.ShapeDtypeStruct((NSEG, 1), jnp.float32),
    )(sums4, cnt2, W1, b1r, W2, b2r)


@jax.jit
def kernel(x, batch, W1, b1, W2, b2):
    xp = (x.reshape(NGRP, 8, CT, LANES)
          .transpose(0, 2, 1, 3)
          .reshape(NGRP * CT * 8, LANES))
    batch2 = batch.astype(jnp.int32).reshape(NCH, CHUNK)
    pidx = (batch2.reshape(NCH, GPC, 1, 8)
            + (NSEG * jnp.arange(CT, dtype=jnp.int32)).reshape(1, 1, CT, 1))
    pidx3 = pidx.reshape(NCH, 2, CHUNK * CT // 2)
    ones = jnp.ones((CHUNK, CW), jnp.float32)
    zs = jnp.zeros((ARS, LANES), jnp.float32)
    zc = jnp.zeros((RS, CW), jnp.float32)
    sums2, cnt2 = _sc_segment_sum(xp, batch2, pidx3, ones, zs, zc)
    sums4 = sums2.reshape(NC, CT, NSEG, LANES)
    W2p = jnp.pad(W2, ((0, 7), (0, 0)))
    out = _tc_head(sums4, cnt2, W1, b1.reshape(1, 128), W2p, b2.reshape(1, 1))
    return out[:, 0]


# speedup vs baseline: 1.2832x; 1.2229x over previous
import functools

import jax
import jax.numpy as jnp
from jax import lax
from jax.experimental import pallas as pl
from jax.experimental.pallas import tpu as pltpu
from jax.experimental.pallas import tpu_sc as plsc

NSEG = 256
NROWS = 100000
D = 768
LANES = 128
CT = D // LANES
NPIECE = NROWS * CT
NC, NS = 2, 16
NW = NC * NS
CHUNK = 32
PIECES = CHUNK * CT
HALF = PIECES // 2
NCH = NROWS // CHUNK
NKMAX = -(-NCH // NW)
NFULL = NCH - NW * (NKMAX - 1)
NBUF = 3
CW = 16
ACC = NSEG * CT
ARS = ACC // NS
RS = NSEG // NS


def _sc_segment_sum(xp, pidx3, batch3, ones, zsum, zcnt):
    mesh = plsc.VectorSubcoreMesh(
        core_axis_name="c", subcore_axis_name="s",
        num_cores=NC, num_subcores=NS)

    @functools.partial(
        pl.kernel,
        out_type=[
            jax.ShapeDtypeStruct((NC, ACC, LANES), jnp.float32),
            jax.ShapeDtypeStruct((NC, NSEG, CW), jnp.float32),
        ],
        mesh=mesh,
        scratch_types=[
            pltpu.VMEM((NKMAX, 2, HALF), jnp.int32),
            pltpu.VMEM((NKMAX, CHUNK), jnp.int32),
            pltpu.VMEM((NBUF, PIECES, LANES), jnp.float32),
            pltpu.VMEM((CHUNK, CW), jnp.float32),
            pltpu.VMEM_SHARED((ACC, LANES), jnp.float32),
            pltpu.VMEM_SHARED((NSEG, CW), jnp.float32),
            pltpu.SemaphoreType.DMA((NBUF,)),
            pltpu.SemaphoreType.DMA((NBUF,)),
            pltpu.SemaphoreType.DMA((NBUF,)),
        ],
        compiler_params=pltpu.CompilerParams(use_tc_tiling_on_sc=False),
    )
    def body(xp_hbm, pidx_hbm, b3_hbm, ones_hbm, zs_hbm, zc_hbm,
             sums_out, cnt_out,
             pidx_v, idx_v, bufs, ones_v, acc_s, acc_c, gsem, ssem, csem):
        c = lax.axis_index("c")
        s = lax.axis_index("s")
        wid = c * NS + s
        nk = jnp.where(wid < NFULL, NKMAX, NKMAX - 1)

        pltpu.sync_copy(zs_hbm, acc_s.at[pl.ds(s * ARS, ARS)])
        pltpu.sync_copy(zc_hbm, acc_c.at[pl.ds(s * RS, RS)])
        pltpu.sync_copy(pidx_hbm.at[wid], pidx_v)
        pltpu.sync_copy(b3_hbm.at[wid], idx_v)
        pltpu.sync_copy(ones_hbm, ones_v)
        plsc.subcore_barrier()

        def start_gather(j, b):
            p0 = (j * NW + wid) * PIECES
            pltpu.async_copy(xp_hbm.at[pl.ds(p0, PIECES)],
                             bufs.at[b], gsem.at[b])

        for j0 in range(NBUF - 2):
            start_gather(j0, j0)

        def chunk_body(k, carry):
            b = lax.rem(k, NBUF)
            pltpu.make_async_copy(xp_hbm.at[pl.ds(0, PIECES)],
                                  bufs.at[b], gsem.at[b]).wait()
            for h in range(2):
                pltpu.async_copy(bufs.at[b, pl.ds(h * HALF, HALF)],
                                 acc_s.at[pidx_v.at[k, h]], ssem.at[b],
                                 add=True)
            pltpu.async_copy(ones_v, acc_c.at[idx_v.at[k]], csem.at[b],
                             add=True)
            j = k + (NBUF - 2)
            bj = lax.rem(j, NBUF)

            @pl.when(j < nk)
            def _():
                @pl.when(j >= NBUF)
                def _():
                    for h in range(2):
                        pltpu.make_async_copy(
                            bufs.at[bj, pl.ds(h * HALF, HALF)],
                            acc_s.at[pidx_v.at[0, 0]], ssem.at[bj]).wait()
                    pltpu.make_async_copy(ones_v, acc_c.at[idx_v.at[0]],
                                          csem.at[bj]).wait()
                start_gather(j, bj)

            return carry

        lax.fori_loop(0, nk, chunk_body, 0)
        for b in range(NBUF):
            for h in range(2):
                pltpu.make_async_copy(bufs.at[b, pl.ds(h * HALF, HALF)],
                                      acc_s.at[pidx_v.at[0, 0]],
                                      ssem.at[b]).wait()
            pltpu.make_async_copy(ones_v, acc_c.at[idx_v.at[0]],
                                  csem.at[b]).wait()
        plsc.subcore_barrier()

        pltpu.sync_copy(acc_s.at[pl.ds(s * ARS, ARS)],
                        sums_out.at[c, pl.ds(s * ARS, ARS)])
        pltpu.sync_copy(acc_c.at[pl.ds(s * RS, RS)],
                        cnt_out.at[c, pl.ds(s * RS, RS)])

    return body(xp, pidx3, batch3, ones, zsum, zcnt)


def _tc_head(sums2, cnt2, W1, b1r, W2, b2r):
    def body(s_ref, c_ref, w1_ref, b1_ref, w2_ref, b2_ref, out_ref):
        sums = s_ref[0] + s_ref[1]
        cnt = c_ref[0, :, 0:1] + c_ref[1, :, 0:1]
        pooled = sums / jnp.clip(cnt, 1.0, None)
        h = lax.dot_general(
            pooled, w1_ref[...],
            dimension_numbers=(((1,), (1,)), ((), ())),
            preferred_element_type=jnp.float32,
            precision=lax.Precision.HIGHEST)
        h = jnp.maximum(h + b1_ref[...], 0.0)
        o = lax.dot_general(
            h, w2_ref[...],
            dimension_numbers=(((1,), (1,)), ((), ())),
            preferred_element_type=jnp.float32,
            precision=lax.Precision.HIGHEST)
        out_ref[...] = o[:, 0:1] + b2_ref[0, 0]

    return pl.pallas_call(
        body,
        in_specs=[
            pl.BlockSpec(memory_space=pltpu.VMEM),
            pl.BlockSpec(memory_space=pltpu.VMEM),
            pl.BlockSpec(memory_space=pltpu.VMEM),
            pl.BlockSpec(memory_space=pltpu.VMEM),
            pl.BlockSpec(memory_space=pltpu.VMEM),
            pl.BlockSpec(memory_space=pltpu.SMEM),
        ],
        out_shape=jax.ShapeDtypeStruct((NSEG, 1), jnp.float32),
    )(sums2, cnt2, W1, b1r, W2, b2r)


@jax.jit
def kernel(x, batch, W1, b1, W2, b2):
    xp = (x.reshape(NROWS // 8, 8, CT, LANES)
          .transpose(0, 2, 1, 3)
          .reshape(NPIECE, LANES))
    ids = batch.astype(jnp.int32)
    b2g = ids.reshape(NROWS // 8, 8)
    pidx = (b2g[:, None, :] * CT
            + jnp.arange(CT, dtype=jnp.int32)[None, :, None])
    pidx = pidx.reshape(NCH, PIECES)
    pidx = jnp.pad(pidx, ((0, NW * NKMAX - NCH), (0, 0)))
    pidx3 = pidx.reshape(NKMAX, NW, 2, HALF).transpose(1, 0, 2, 3)
    ids2 = jnp.pad(ids.reshape(NCH, CHUNK), ((0, NW * NKMAX - NCH), (0, 0)))
    batch3 = ids2.reshape(NKMAX, NW, CHUNK).transpose(1, 0, 2)
    ones = jnp.ones((CHUNK, CW), jnp.float32)
    zs = jnp.zeros((ARS, LANES), jnp.float32)
    zc = jnp.zeros((RS, CW), jnp.float32)
    sums2, cnt2 = _sc_segment_sum(xp, pidx3, batch3, ones, zs, zc)
    sums2 = sums2.reshape(NC, NSEG, D)
    W2p = jnp.pad(W2, ((0, 7), (0, 0)))
    out = _tc_head(sums2, cnt2, W1, b1.reshape(1, 128), W2p, b2.reshape(1, 1))
    return out[:, 0]
